# Initial kernel scaffold; baseline (speedup 1.0000x reference)
import jax
import jax.numpy as jnp
from jax.experimental import pallas as pl


def kernel(xyz, tables):
    raise NotImplementedError("write your pallas kernel here")


# speedup vs baseline: 19.6766x; 19.6766x over previous
import math

import jax
import jax.numpy as jnp
import numpy as np
from jax import lax
from jax.experimental import pallas as pl
from jax.experimental.pallas import tpu as pltpu
from jax.experimental.pallas import tpu_sc as plsc

_L = 16
_T = 2 ** 19
_F = 2
_N_MIN = 16
_N_MAX = 2048
_BB_MIN = -1.0
_GROWTH = math.exp((math.log(_N_MAX) - math.log(_N_MIN)) / (_L - 1))
_RES = [int(math.floor(_N_MIN * (_GROWTH ** i))) for i in range(_L)]
_CELL = [np.float32(2.0 / r) for r in _RES]
_PI1 = np.uint32(2654435761)
_PI2 = np.uint32(805459861)
_MASK = np.uint32(_T - 1)

_N = 262144
_NW = 32
_P = _N // _NW
_C = 128
_NCH = _P // _C
_E = 2 * 8 * _L * _C


def _sc_body(x_hbm, y_hbm, z_hbm, tab_hbm, out_hbm,
             xv, yv, zv, idxv, rowsv, outv, sem):
    wid = lax.axis_index("s") * 2 + lax.axis_index("c")
    lane = lax.iota(jnp.int32, 16)
    halfl = lax.shift_right_logical(lane, 1)
    feat = lane & 1
    pos_e = 2 * lane

    def chunk_body(ch, carry):
        base = wid * _P + ch * _C
        pltpu.sync_copy(x_hbm.at[pl.ds(base, _C)], xv)
        pltpu.sync_copy(y_hbm.at[pl.ds(base, _C)], yv)
        pltpu.sync_copy(z_hbm.at[pl.ds(base, _C)], zv)

        def idx_body(g, c2):
            col = g * 16
            px = xv[pl.ds(col, 16)]
            py = yv[pl.ds(col, 16)]
            pz = zv[pl.ds(col, 16)]
            for i in range(_L):
                cell = _CELL[i]
                ux = ((px - jnp.float32(_BB_MIN)) / cell).astype(jnp.int32).astype(jnp.uint32)
                uy = ((py - jnp.float32(_BB_MIN)) / cell).astype(jnp.int32).astype(jnp.uint32)
                uz = ((pz - jnp.float32(_BB_MIN)) / cell).astype(jnp.int32).astype(jnp.uint32)
                hx = (ux, ux + np.uint32(1))
                hy = (uy * _PI1, (uy + np.uint32(1)) * _PI1)
                hz = (uz * _PI2, (uz + np.uint32(1)) * _PI2)
                for a in range(2):
                    for b in range(2):
                        hxy = hx[a] ^ hy[b]
                        for c in range(2):
                            h = ((hxy ^ hz[c]) & _MASK).astype(jnp.int32)
                            blk = i * 8 + 4 * a + 2 * b + c
                            e2 = 2 * (h + jnp.int32(i * _T))
                            off = blk * (2 * _C) + 2 * col
                            plsc.store_scatter(idxv, [off + pos_e, ], e2)
                            plsc.store_scatter(idxv, [off + pos_e + 1, ], e2 + 1)
            return c2

        lax.fori_loop(0, _C // 16, idx_body, 0)

        pltpu.async_copy(tab_hbm.at[idxv], rowsv, sem)
        pltpu.make_async_copy(tab_hbm.at[idxv], rowsv, sem).wait()

        def interp_body(g, c2):
            p0 = g * 8
            dupi = p0 + halfl
            xd = plsc.load_gather(xv, [dupi])
            yd = plsc.load_gather(yv, [dupi])
            zd = plsc.load_gather(zv, [dupi])
            for i in range(_L):
                cell = _CELL[i]

                def dcoord(pd):
                    t = (pd - jnp.float32(_BB_MIN)) / cell
                    mv = t.astype(jnp.int32).astype(jnp.float32) * cell + jnp.float32(_BB_MIN)
                    den = (mv + cell) - mv
                    return (pd - mv) / den

                dx = dcoord(xd)
                dy = dcoord(yd)
                dz = dcoord(zd)
                e = []
                for j in range(8):
                    off = (i * 8 + j) * (2 * _C) + 2 * p0
                    e.append(rowsv[pl.ds(off, 16)])
                omx = jnp.float32(1.0) - dx
                c00 = e[0] * omx + e[4] * dx
                c01 = e[1] * omx + e[5] * dx
                c10 = e[2] * omx + e[6] * dx
                c11 = e[3] * omx + e[7] * dx
                omy = jnp.float32(1.0) - dy
                c0 = c00 * omy + c10 * dy
                c1 = c01 * omy + c11 * dy
                c = c0 * (jnp.float32(1.0) - dz) + c1 * dz
                plsc.store_scatter(outv, [dupi, 2 * i + feat], c)
            return c2

        lax.fori_loop(0, _C // 8, interp_body, 0)

        pltpu.sync_copy(outv, out_hbm.at[pl.ds(base, _C), :])
        return carry

    lax.fori_loop(0, _NCH, chunk_body, 0)


_launch = pl.kernel(
    _sc_body,
    out_type=jax.ShapeDtypeStruct((_N, 2 * _L), jnp.float32),
    mesh=plsc.VectorSubcoreMesh(core_axis_name="c", subcore_axis_name="s"),
    compiler_params=pltpu.CompilerParams(
        needs_layout_passes=False, use_tc_tiling_on_sc=False),
    scratch_types=[
        pltpu.VMEM((_C,), jnp.float32),
        pltpu.VMEM((_C,), jnp.float32),
        pltpu.VMEM((_C,), jnp.float32),
        pltpu.VMEM((_E,), jnp.int32),
        pltpu.VMEM((_E,), jnp.float32),
        pltpu.VMEM((_C, 2 * _L), jnp.float32),
        pltpu.SemaphoreType.DMA,
    ],
)


def kernel(xyz, tables):
    xyzt = xyz.T
    tab = tables.reshape(_L * _T * _F)
    return _launch(xyzt[0], xyzt[1], xyzt[2], tab)


# speedup vs baseline: 20.0166x; 1.0173x over previous
import math

import jax
import jax.numpy as jnp
import numpy as np
from jax import lax
from jax.experimental import pallas as pl
from jax.experimental.pallas import tpu as pltpu
from jax.experimental.pallas import tpu_sc as plsc

_L = 16
_T = 2 ** 19
_F = 2
_N_MIN = 16
_N_MAX = 2048
_BB_MIN = -1.0
_GROWTH = math.exp((math.log(_N_MAX) - math.log(_N_MIN)) / (_L - 1))
_RES = [int(math.floor(_N_MIN * (_GROWTH ** i))) for i in range(_L)]
_CELL = [np.float32(2.0 / r) for r in _RES]
_PI1 = np.uint32(2654435761)
_PI2 = np.uint32(805459861)
_MASK = np.uint32(_T - 1)

_N = 262144
_NW = 32
_P = _N // _NW
_C = 128
_NCH = _P // _C
_E = 2 * 8 * _L * _C


def _sc_body(x_hbm, y_hbm, z_hbm, tab_hbm, out_hbm,
             xv, yv, zv, idxv, rowsv, outv, sem):
    wid = lax.axis_index("s") * 2 + lax.axis_index("c")
    lane = lax.iota(jnp.int32, 16)
    halfl = lax.shift_right_logical(lane, 1)
    feat = lane & 1
    pos_e = 2 * lane

    def chunk_body(ch, carry):
        base = wid * _P + ch * _C
        pltpu.sync_copy(x_hbm.at[pl.ds(base, _C)], xv)
        pltpu.sync_copy(y_hbm.at[pl.ds(base, _C)], yv)
        pltpu.sync_copy(z_hbm.at[pl.ds(base, _C)], zv)

        def mk_idx_body(i):
            cell = _CELL[i]

            def idx_body(g, c2):
                col = g * 16
                px = xv[pl.ds(col, 16)]
                py = yv[pl.ds(col, 16)]
                pz = zv[pl.ds(col, 16)]
                ux = ((px - jnp.float32(_BB_MIN)) / cell).astype(jnp.int32).astype(jnp.uint32)
                uy = ((py - jnp.float32(_BB_MIN)) / cell).astype(jnp.int32).astype(jnp.uint32)
                uz = ((pz - jnp.float32(_BB_MIN)) / cell).astype(jnp.int32).astype(jnp.uint32)
                hx = (ux, ux + np.uint32(1))
                hy = (uy * _PI1, (uy + np.uint32(1)) * _PI1)
                hz = (uz * _PI2, (uz + np.uint32(1)) * _PI2)
                for a in range(2):
                    for b in range(2):
                        hxy = hx[a] ^ hy[b]
                        for c in range(2):
                            h = ((hxy ^ hz[c]) & _MASK).astype(jnp.int32)
                            blk = i * 8 + 4 * a + 2 * b + c
                            e2 = 2 * (h + jnp.int32(i * _T))
                            off = blk * (2 * _C) + 2 * col
                            plsc.store_scatter(idxv, [off + pos_e, ], e2)
                            plsc.store_scatter(idxv, [off + pos_e + 1, ], e2 + 1)
                return c2

            return idx_body

        seg = 8 * 2 * _C
        for i in range(_L):
            lax.fori_loop(0, _C // 16, mk_idx_body(i), 0)
            pltpu.async_copy(tab_hbm.at[idxv.at[pl.ds(i * seg, seg)]],
                             rowsv.at[pl.ds(i * seg, seg)], sem)

        pltpu.make_async_copy(tab_hbm.at[idxv], rowsv, sem).wait()

        def interp_body(g, c2):
            p0 = g * 8
            dupi = p0 + halfl
            xd = plsc.load_gather(xv, [dupi])
            yd = plsc.load_gather(yv, [dupi])
            zd = plsc.load_gather(zv, [dupi])
            for i in range(_L):
                cell = _CELL[i]

                def dcoord(pd):
                    t = (pd - jnp.float32(_BB_MIN)) / cell
                    mv = t.astype(jnp.int32).astype(jnp.float32) * cell + jnp.float32(_BB_MIN)
                    den = (mv + cell) - mv
                    return (pd - mv) / den

                dx = dcoord(xd)
                dy = dcoord(yd)
                dz = dcoord(zd)
                e = []
                for j in range(8):
                    off = (i * 8 + j) * (2 * _C) + 2 * p0
                    e.append(rowsv[pl.ds(off, 16)])
                omx = jnp.float32(1.0) - dx
                c00 = e[0] * omx + e[4] * dx
                c01 = e[1] * omx + e[5] * dx
                c10 = e[2] * omx + e[6] * dx
                c11 = e[3] * omx + e[7] * dx
                omy = jnp.float32(1.0) - dy
                c0 = c00 * omy + c10 * dy
                c1 = c01 * omy + c11 * dy
                c = c0 * (jnp.float32(1.0) - dz) + c1 * dz
                plsc.store_scatter(outv, [dupi, 2 * i + feat], c)
            return c2

        lax.fori_loop(0, _C // 8, interp_body, 0)

        pltpu.sync_copy(outv, out_hbm.at[pl.ds(base, _C), :])
        return carry

    lax.fori_loop(0, _NCH, chunk_body, 0)


_launch = pl.kernel(
    _sc_body,
    out_type=jax.ShapeDtypeStruct((_N, 2 * _L), jnp.float32),
    mesh=plsc.VectorSubcoreMesh(core_axis_name="c", subcore_axis_name="s"),
    compiler_params=pltpu.CompilerParams(
        needs_layout_passes=False, use_tc_tiling_on_sc=False),
    scratch_types=[
        pltpu.VMEM((_C,), jnp.float32),
        pltpu.VMEM((_C,), jnp.float32),
        pltpu.VMEM((_C,), jnp.float32),
        pltpu.VMEM((_E,), jnp.int32),
        pltpu.VMEM((_E,), jnp.float32),
        pltpu.VMEM((_C, 2 * _L), jnp.float32),
        pltpu.SemaphoreType.DMA,
    ],
)


def kernel(xyz, tables):
    xyzt = xyz.T
    tab = tables.reshape(_L * _T * _F)
    return _launch(xyzt[0], xyzt[1], xyzt[2], tab)
